# Initial kernel scaffold; baseline (speedup 1.0000x reference)
import jax
import jax.numpy as jnp
from jax.experimental import pallas as pl


def kernel(attn_bias, spatial_pos, d2_dist, a3_dist, edge_data, edge_path, edge_padding_mask, graph, node_data, spatial_pos_table, gt_vd, d2_W, d2_b, a3_W, a3_b, curv_W1, curv_b1, curv_W2, curv_b2, nc_W, nc_b, edge_dis_weight):
    raise NotImplementedError("write your pallas kernel here")


# speedup vs baseline: 18.7086x; 18.7086x over previous
import functools

import jax
import jax.numpy as jnp
from jax import lax
from jax.experimental import pallas as pl
from jax.experimental.pallas import tpu as pltpu
from jax.experimental.pallas import tpu_sc as plsc

B, N, L, H, MD = 16, 128, 512, 32, 5
NN = N * N
BNN = B * NN
NE = B * L
LP = L + 8
TRORS = B * LP
HM = MD * H
NSPT = 512
NC, NS = 2, 16
NW = NC * NS
POS_W = BNN // NW
G = 64
NCHUNK = POS_W // G
EPW = NE // NW
TW = 128

_MESH = dict(core_axis_name="c", subcore_axis_name="s")


def _precompute_body(ed_ref, w1_ref, b1_ref, w2_ref, bc_ref, w5_ref, nd_ref,
                     ncw_ref, hm_ref, projm_ref):
    w5 = w5_ref[...]
    w2c = jnp.dot(w2_ref[...], w5, preferred_element_type=jnp.float32)
    biasc = jnp.dot(bc_ref[...], w5, preferred_element_type=jnp.float32)
    a = jnp.dot(ed_ref[...], w1_ref[...], preferred_element_type=jnp.float32)
    a = jnp.maximum(a + b1_ref[...], 0.0)
    hm_ref[...] = jnp.dot(a, w2c, preferred_element_type=jnp.float32) + biasc
    ncm = jnp.dot(ncw_ref[...], w5, preferred_element_type=jnp.float32)
    projm_ref[:, 0:HM] = jnp.dot(nd_ref[...], ncm,
                                 preferred_element_type=jnp.float32)


def _precompute(ed, w1, b1, w2, bc, w5, nd, ncw):
    return pl.pallas_call(
        _precompute_body,
        out_shape=(jax.ShapeDtypeStruct((NE, HM), jnp.float32),
                   jax.ShapeDtypeStruct((B * N, 2 * TW), jnp.float32)),
    )(ed, w1, b1, w2, bc, w5, nd, ncw)


def _table_body(hm_hbm, projm_hbm, src_hbm, dst_hbm, t_hbm,
                hb, sb, db, sib, dib, tb, zr, sem):
    w = lax.axis_index("s") * NC + lax.axis_index("c")
    b = w // 2
    eh = 64

    def half(k, _):
        e0 = w * EPW + k * eh
        cps = [pltpu.async_copy(hm_hbm.at[pl.ds(e0, eh), :], hb, sem),
               pltpu.async_copy(src_hbm.at[pl.ds(e0, eh)], sib, sem),
               pltpu.async_copy(dst_hbm.at[pl.ds(e0, eh)], dib, sem)]
        for c in cps:
            c.wait()
        gs = pltpu.async_copy(projm_hbm.at[sib], sb, sem)
        gd = pltpu.async_copy(projm_hbm.at[dib], db, sem)
        gs.wait()
        gd.wait()

        def row(p, _):
            for m in range(MD):
                for h in range(2):
                    dsrc = pl.ds(m * H + h * 16, 16)
                    tb[p * MD + m, pl.ds(h * 16, 16)] = (
                        hb[p, dsrc] + sb[p, dsrc] + db[p, dsrc])
            return 0

        lax.fori_loop(0, eh, row, 0)
        pltpu.sync_copy(tb, t_hbm.at[pl.ds((e0 + b * 8) * MD, eh * MD), :])
        return 0

    lax.fori_loop(0, EPW // eh, half, 0)

    @pl.when(w % 2 == 1)
    def _zero_rows():
        def zrow(p, _):
            for q in range(TW // 16):
                zr[p, pl.ds(q * 16, 16)] = jnp.zeros((16,), jnp.float32)
            return 0

        lax.fori_loop(0, 8 * MD, zrow, 0)
        pltpu.sync_copy(zr, t_hbm.at[pl.ds((b * LP + L) * MD, 8 * MD), :])


def _build_table(hm, projm, src, dst):
    f = pl.kernel(
        _table_body,
        out_type=jax.ShapeDtypeStruct((TRORS * MD, TW), jnp.float32),
        mesh=plsc.VectorSubcoreMesh(**_MESH),
        scratch_types=[
            pltpu.VMEM((64, HM), jnp.float32),
            pltpu.VMEM((64, 2 * TW), jnp.float32),
            pltpu.VMEM((64, 2 * TW), jnp.float32),
            pltpu.VMEM((64,), jnp.int32),
            pltpu.VMEM((64,), jnp.int32),
            pltpu.VMEM((64 * MD, TW), jnp.float32),
            pltpu.VMEM((8 * MD, TW), jnp.float32),
            pltpu.SemaphoreType.DMA,
        ],
    )
    return f(hm, projm, src, dst)


def _gather_body(idxe_hbm, sp_hbm, tbl_hbm, ext_hbm, acc_hbm,
                 eb, exb, ob, ib, spb, sem):
    w = lax.axis_index("s") * NC + lax.axis_index("c")

    def chunk(c, _):
        base = w * POS_W + c * G
        cps = [pltpu.async_copy(idxe_hbm.at[pl.ds(m * BNN + base, G)], ib[m], sem)
               for m in range(MD)]
        cps.append(pltpu.async_copy(sp_hbm.at[pl.ds(base, G)], spb, sem))
        for cp in cps:
            cp.wait()
        gs = [pltpu.async_copy(tbl_hbm.at[ib[m]], eb[m], sem) for m in range(MD)]
        gs.append(pltpu.async_copy(ext_hbm.at[spb], exb, sem))
        for g in gs:
            g.wait()

        def row(p, _):
            lo = pl.ds(0, 16)
            hi = pl.ds(16, 16)
            r = exb[p, pl.ds(32, 16)]
            e0 = ((eb[0][p, lo] + eb[1][p, lo]) + (eb[2][p, lo] + eb[3][p, lo])
                  + eb[4][p, lo])
            e1 = ((eb[0][p, hi] + eb[1][p, hi]) + (eb[2][p, hi] + eb[3][p, hi])
                  + eb[4][p, hi])
            ob[p, lo] = exb[p, lo] + r * e0
            ob[p, hi] = exb[p, hi] + r * e1
            return 0

        lax.fori_loop(0, G, row, 0)
        pltpu.sync_copy(ob, acc_hbm.at[pl.ds(base, G), :])
        return 0

    lax.fori_loop(0, NCHUNK, chunk, 0)


def _gather_acc(idx_e, sp_flat, tbl32, spt_ext):
    f = pl.kernel(
        _gather_body,
        out_type=jax.ShapeDtypeStruct((BNN, H), jnp.float32),
        mesh=plsc.VectorSubcoreMesh(**_MESH),
        scratch_types=[
            [pltpu.VMEM((G, TW), jnp.float32) for _ in range(MD)],
            pltpu.VMEM((G, TW), jnp.float32),
            pltpu.VMEM((G, H), jnp.float32),
            [pltpu.VMEM((G,), jnp.int32) for _ in range(MD)],
            pltpu.VMEM((G,), jnp.int32),
            pltpu.SemaphoreType.DMA,
        ],
    )
    return f(idx_e, sp_flat, tbl32, spt_ext)


def _assemble_body(ab_ref, x2_ref, x3_ref, acc_ref, wc_ref, bias_ref, gt_ref,
                   out_ref):
    xc = jnp.concatenate(
        [x2_ref[0].astype(jnp.bfloat16), x3_ref[0].astype(jnp.bfloat16),
         acc_ref[0].astype(jnp.bfloat16)], axis=1)
    wc = wc_ref[...].astype(jnp.bfloat16)
    st = lax.dot_general(wc, xc, (((0,), (1,)), ((), ())),
                         preferred_element_type=jnp.float32)
    st = st + bias_ref[...].reshape(H, 1)
    st = st.reshape(H, N, N)
    ab2 = ab_ref[0] * 2.0
    t = gt_ref[...].reshape(H, 1)
    top = ab2[0:1, :] + t
    left = ab2[1:, 0:1].reshape(1, N) + t
    inter = st + ab2[1:, 1:][None, :, :]
    out_ref[0, :, 0, :] = top
    out_ref[0, :, 1:, :] = jnp.concatenate([left[:, :, None], inter], axis=2)


def _assemble(ab, x2, x3, acc3, wc, bias2, gt):
    return pl.pallas_call(
        _assemble_body,
        grid=(B,),
        in_specs=[
            pl.BlockSpec((1, N + 1, N + 1), lambda b: (b, 0, 0)),
            pl.BlockSpec((1, NN, H), lambda b: (b, 0, 0)),
            pl.BlockSpec((1, NN, H), lambda b: (b, 0, 0)),
            pl.BlockSpec((1, NN, H), lambda b: (b, 0, 0)),
            pl.BlockSpec((3 * H, H), lambda b: (0, 0)),
            pl.BlockSpec((1, H), lambda b: (0, 0)),
            pl.BlockSpec((1, H), lambda b: (0, 0)),
        ],
        out_specs=pl.BlockSpec((1, H, N + 1, N + 1), lambda b: (b, 0, 0, 0)),
        out_shape=jax.ShapeDtypeStruct((B, H, N + 1, N + 1), jnp.float32),
        compiler_params=pltpu.CompilerParams(
            vmem_limit_bytes=100 * 1024 * 1024),
    )(ab, x2, x3, acc3, wc, bias2, gt)


def kernel(attn_bias, spatial_pos, d2_dist, a3_dist, edge_data, edge_path,
           edge_padding_mask, graph, node_data, spatial_pos_table, gt_vd,
           d2_W, d2_b, a3_W, a3_b, curv_W1, curv_b1, curv_W2, curv_b2,
           nc_W, nc_b, edge_dis_weight):
    f32 = jnp.float32
    w5 = edge_dis_weight.reshape(-1, H, H)[:MD]
    w5cat = jnp.concatenate([w5[m] for m in range(MD)], axis=1)
    bc = (curv_b2 + nc_b).reshape(1, H)
    ed = edge_data.reshape(NE, 7)

    hm, projm = _precompute(ed, curv_W1, curv_b1.reshape(1, 64), curv_W2, bc,
                            w5cat, node_data, nc_W)

    src = graph[0].astype(jnp.int32)
    dst = graph[1].astype(jnp.int32)
    tbl32 = _build_table(hm, projm, src, dst)

    v = jnp.arange(NSPT)
    s = jnp.clip(jnp.where(v == 0, 1, jnp.where(v > 1, v - 1, v)), 0, MD)
    rec = 1.0 / s.astype(f32)
    spt_ext = jnp.concatenate(
        [spatial_pos_table, jnp.repeat(rec[:, None], 16, axis=1),
         jnp.zeros((NSPT, TW - H - 16), f32)], axis=1)

    ep = edge_path.reshape(B, NN, MD).astype(jnp.int32)
    boff = (jnp.arange(B, dtype=jnp.int32) * (LP * MD))[:, None, None]
    gidx = ep * MD + boff + jnp.arange(MD, dtype=jnp.int32)[None, None, :]
    idx_e = gidx.reshape(BNN, MD).T.reshape(MD * BNN)
    sp_flat = spatial_pos.reshape(BNN).astype(jnp.int32)

    acc = _gather_acc(idx_e, sp_flat, tbl32, spt_ext)

    wc = jnp.concatenate([d2_W, a3_W, jnp.eye(H, dtype=f32)], axis=0)
    bias2 = (d2_b + a3_b).reshape(1, H)
    out = _assemble(attn_bias, d2_dist.reshape(B, NN, H),
                    a3_dist.reshape(B, NN, H), acc.reshape(B, NN, H),
                    wc, bias2, gt_vd)
    return out


# speedup vs baseline: 23.6603x; 1.2647x over previous
import functools

import jax
import jax.numpy as jnp
from jax import lax
from jax.experimental import pallas as pl
from jax.experimental.pallas import tpu as pltpu
from jax.experimental.pallas import tpu_sc as plsc

B, N, L, H, MD = 16, 128, 512, 32, 5
NN = N * N
BNN = B * NN
NE = B * L
LP = L + 8
TRORS = B * LP
HM = MD * H
NSPT = 512
NC, NS = 2, 16
NW = NC * NS
POS_W = BNN // NW
G = 64
NCHUNK = POS_W // G
EPW = NE // NW
TW = 128

_MESH = dict(core_axis_name="c", subcore_axis_name="s")


def _precompute_body(ed_ref, w1_ref, b1_ref, w2_ref, bc_ref, w5_ref, nd_ref,
                     ncw_ref, hm_ref, projm_ref):
    w5 = w5_ref[...]
    w2c = jnp.dot(w2_ref[...], w5, preferred_element_type=jnp.float32)
    biasc = jnp.dot(bc_ref[...], w5, preferred_element_type=jnp.float32)
    a = jnp.dot(ed_ref[...], w1_ref[...], preferred_element_type=jnp.float32)
    a = jnp.maximum(a + b1_ref[...], 0.0)
    hm_ref[...] = jnp.dot(a, w2c, preferred_element_type=jnp.float32) + biasc
    ncm = jnp.dot(ncw_ref[...], w5, preferred_element_type=jnp.float32)
    projm_ref[:, 0:HM] = jnp.dot(nd_ref[...], ncm,
                                 preferred_element_type=jnp.float32)


def _precompute(ed, w1, b1, w2, bc, w5, nd, ncw):
    return pl.pallas_call(
        _precompute_body,
        out_shape=(jax.ShapeDtypeStruct((NE, HM), jnp.float32),
                   jax.ShapeDtypeStruct((B * N, 2 * TW), jnp.float32)),
    )(ed, w1, b1, w2, bc, w5, nd, ncw)


def _table_body(hm_hbm, projm_hbm, src_hbm, dst_hbm, t_hbm,
                hb, sb, db, sib, dib, tb, zr, sem):
    w = lax.axis_index("s") * NC + lax.axis_index("c")
    b = w // 2
    eh = 64

    def half(k, _):
        e0 = w * EPW + k * eh
        cps = [pltpu.async_copy(hm_hbm.at[pl.ds(e0, eh), :], hb, sem),
               pltpu.async_copy(src_hbm.at[pl.ds(e0, eh)], sib, sem),
               pltpu.async_copy(dst_hbm.at[pl.ds(e0, eh)], dib, sem)]
        for c in cps:
            c.wait()
        gs = pltpu.async_copy(projm_hbm.at[sib], sb, sem)
        gd = pltpu.async_copy(projm_hbm.at[dib], db, sem)
        gs.wait()
        gd.wait()

        def row(p, _):
            for m in range(MD):
                for h in range(2):
                    dsrc = pl.ds(m * H + h * 16, 16)
                    tb[p * MD + m, pl.ds(h * 16, 16)] = (
                        hb[p, dsrc] + sb[p, dsrc] + db[p, dsrc])
            return 0

        lax.fori_loop(0, eh, row, 0)
        pltpu.sync_copy(tb, t_hbm.at[pl.ds((e0 + b * 8) * MD, eh * MD), :])
        return 0

    lax.fori_loop(0, EPW // eh, half, 0)

    @pl.when(w % 2 == 1)
    def _zero_rows():
        def zrow(p, _):
            for q in range(TW // 16):
                zr[p, pl.ds(q * 16, 16)] = jnp.zeros((16,), jnp.float32)
            return 0

        lax.fori_loop(0, 8 * MD, zrow, 0)
        pltpu.sync_copy(zr, t_hbm.at[pl.ds((b * LP + L) * MD, 8 * MD), :])


def _build_table(hm, projm, src, dst):
    f = pl.kernel(
        _table_body,
        out_type=jax.ShapeDtypeStruct((TRORS * MD, TW), jnp.float32),
        mesh=plsc.VectorSubcoreMesh(**_MESH),
        scratch_types=[
            pltpu.VMEM((64, HM), jnp.float32),
            pltpu.VMEM((64, 2 * TW), jnp.float32),
            pltpu.VMEM((64, 2 * TW), jnp.float32),
            pltpu.VMEM((64,), jnp.int32),
            pltpu.VMEM((64,), jnp.int32),
            pltpu.VMEM((64 * MD, TW), jnp.float32),
            pltpu.VMEM((8 * MD, TW), jnp.float32),
            pltpu.SemaphoreType.DMA,
        ],
    )
    return f(hm, projm, src, dst)


def _gather_body(idxe_hbm, sp_hbm, tbl_hbm, ext_hbm, acc_hbm,
                 eb, exb, ob, ib, spb, semi, semg, semo):
    w = lax.axis_index("s") * NC + lax.axis_index("c")
    w0 = w * POS_W

    def idx_cps(c, par):
        base = w0 + c * G
        cps = [pltpu.make_async_copy(idxe_hbm.at[pl.ds(m * BNN + base, G)],
                                     ib[par][m], semi[par]) for m in range(MD)]
        cps.append(pltpu.make_async_copy(sp_hbm.at[pl.ds(base, G)], spb[par],
                                         semi[par]))
        return cps

    def gather_cps(par):
        cps = [pltpu.make_async_copy(tbl_hbm.at[ib[par][m]], eb[par][m],
                                     semg[par]) for m in range(MD)]
        cps.append(pltpu.make_async_copy(ext_hbm.at[spb[par]], exb[par],
                                         semg[par]))
        return cps

    def out_cp(c, par):
        return pltpu.make_async_copy(ob[par],
                                     acc_hbm.at[pl.ds(w0 + c * G, G), :],
                                     semo[par])

    for cp in idx_cps(0, 0):
        cp.start()
    for cp in idx_cps(0, 0):
        cp.wait()
    for cp in gather_cps(0):
        cp.start()
    for cp in idx_cps(1, 1):
        cp.start()

    def pair(it, _):
        for par in range(2):
            c = 2 * it + par

            @pl.when(c + 1 < NCHUNK)
            def _advance():
                for cp in idx_cps(c + 1, 1 - par):
                    cp.wait()
                for cp in gather_cps(1 - par):
                    cp.start()

            for cp in gather_cps(par):
                cp.wait()

            @pl.when(c + 2 < NCHUNK)
            def _prefetch_idx():
                for cp in idx_cps(c + 2, par):
                    cp.start()

            @pl.when(c >= 2)
            def _drain_out():
                out_cp(c - 2, par).wait()

            def row(p, _):
                lo = pl.ds(0, 16)
                hi = pl.ds(16, 16)
                e = eb[par]
                r = exb[par][p, pl.ds(32, 16)]
                e0 = ((e[0][p, lo] + e[1][p, lo]) + (e[2][p, lo] + e[3][p, lo])
                      + e[4][p, lo])
                e1 = ((e[0][p, hi] + e[1][p, hi]) + (e[2][p, hi] + e[3][p, hi])
                      + e[4][p, hi])
                ob[par][p, lo] = exb[par][p, lo] + r * e0
                ob[par][p, hi] = exb[par][p, hi] + r * e1
                return 0

            lax.fori_loop(0, G, row, 0)
            out_cp(c, par).start()
        return 0

    lax.fori_loop(0, NCHUNK // 2, pair, 0)
    out_cp(NCHUNK - 2, 0).wait()
    out_cp(NCHUNK - 1, 1).wait()


def _gather_acc(idx_e, sp_flat, tbl32, spt_ext):
    f = pl.kernel(
        _gather_body,
        out_type=jax.ShapeDtypeStruct((BNN, H), jnp.float32),
        mesh=plsc.VectorSubcoreMesh(**_MESH),
        scratch_types=[
            [[pltpu.VMEM((G, TW), jnp.float32) for _ in range(MD)]
             for _ in range(2)],
            [pltpu.VMEM((G, TW), jnp.float32) for _ in range(2)],
            [pltpu.VMEM((G, H), jnp.float32) for _ in range(2)],
            [[pltpu.VMEM((G,), jnp.int32) for _ in range(MD)]
             for _ in range(2)],
            [pltpu.VMEM((G,), jnp.int32) for _ in range(2)],
            [pltpu.SemaphoreType.DMA for _ in range(2)],
            [pltpu.SemaphoreType.DMA for _ in range(2)],
            [pltpu.SemaphoreType.DMA for _ in range(2)],
        ],
    )
    return f(idx_e, sp_flat, tbl32, spt_ext)


def _assemble_body(ab_ref, x2_ref, x3_ref, acc_ref, wc_ref, bias_ref, gt_ref,
                   out_ref):
    xc = jnp.concatenate(
        [x2_ref[0].astype(jnp.bfloat16), x3_ref[0].astype(jnp.bfloat16),
         acc_ref[0].astype(jnp.bfloat16)], axis=1)
    wc = wc_ref[...].astype(jnp.bfloat16)
    st = lax.dot_general(wc, xc, (((0,), (1,)), ((), ())),
                         preferred_element_type=jnp.float32)
    st = st + bias_ref[...].reshape(H, 1)
    st = st.reshape(H, N, N)
    ab2 = ab_ref[0] * 2.0
    t = gt_ref[...].reshape(H, 1)
    top = ab2[0:1, :] + t
    left = ab2[1:, 0:1].reshape(1, N) + t
    inter = st + ab2[1:, 1:][None, :, :]
    out_ref[0, :, 0, :] = top
    out_ref[0, :, 1:, :] = jnp.concatenate([left[:, :, None], inter], axis=2)


def _assemble(ab, x2, x3, acc3, wc, bias2, gt):
    return pl.pallas_call(
        _assemble_body,
        grid=(B,),
        in_specs=[
            pl.BlockSpec((1, N + 1, N + 1), lambda b: (b, 0, 0)),
            pl.BlockSpec((1, NN, H), lambda b: (b, 0, 0)),
            pl.BlockSpec((1, NN, H), lambda b: (b, 0, 0)),
            pl.BlockSpec((1, NN, H), lambda b: (b, 0, 0)),
            pl.BlockSpec((3 * H, H), lambda b: (0, 0)),
            pl.BlockSpec((1, H), lambda b: (0, 0)),
            pl.BlockSpec((1, H), lambda b: (0, 0)),
        ],
        out_specs=pl.BlockSpec((1, H, N + 1, N + 1), lambda b: (b, 0, 0, 0)),
        out_shape=jax.ShapeDtypeStruct((B, H, N + 1, N + 1), jnp.float32),
        compiler_params=pltpu.CompilerParams(
            vmem_limit_bytes=100 * 1024 * 1024),
    )(ab, x2, x3, acc3, wc, bias2, gt)


def kernel(attn_bias, spatial_pos, d2_dist, a3_dist, edge_data, edge_path,
           edge_padding_mask, graph, node_data, spatial_pos_table, gt_vd,
           d2_W, d2_b, a3_W, a3_b, curv_W1, curv_b1, curv_W2, curv_b2,
           nc_W, nc_b, edge_dis_weight):
    f32 = jnp.float32
    w5 = edge_dis_weight.reshape(-1, H, H)[:MD]
    w5cat = jnp.concatenate([w5[m] for m in range(MD)], axis=1)
    bc = (curv_b2 + nc_b).reshape(1, H)
    ed = edge_data.reshape(NE, 7)

    hm, projm = _precompute(ed, curv_W1, curv_b1.reshape(1, 64), curv_W2, bc,
                            w5cat, node_data, nc_W)

    src = graph[0].astype(jnp.int32)
    dst = graph[1].astype(jnp.int32)
    tbl32 = _build_table(hm, projm, src, dst)

    v = jnp.arange(NSPT)
    s = jnp.clip(jnp.where(v == 0, 1, jnp.where(v > 1, v - 1, v)), 0, MD)
    rec = 1.0 / s.astype(f32)
    spt_ext = jnp.concatenate(
        [spatial_pos_table, jnp.repeat(rec[:, None], 16, axis=1),
         jnp.zeros((NSPT, TW - H - 16), f32)], axis=1)

    ep = edge_path.reshape(B, NN, MD).astype(jnp.int32)
    boff = (jnp.arange(B, dtype=jnp.int32) * (LP * MD))[:, None, None]
    gidx = ep * MD + boff + jnp.arange(MD, dtype=jnp.int32)[None, None, :]
    idx_e = gidx.reshape(BNN, MD).T.reshape(MD * BNN)
    sp_flat = spatial_pos.reshape(BNN).astype(jnp.int32)

    acc = _gather_acc(idx_e, sp_flat, tbl32, spt_ext)

    wc = jnp.concatenate([d2_W, a3_W, jnp.eye(H, dtype=f32)], axis=0)
    bias2 = (d2_b + a3_b).reshape(1, H)
    out = _assemble(attn_bias, d2_dist.reshape(B, NN, H),
                    a3_dist.reshape(B, NN, H), acc.reshape(B, NN, H),
                    wc, bias2, gt_vd)
    return out
